# baseline (speedup 1.0000x reference)
import functools
import math

import jax
import jax.numpy as jnp
from jax import lax
from jax.experimental import pallas as pl
from jax.experimental.pallas import tpu as pltpu
from jax.experimental.pallas import tpu_sc as plsc

_D = 64
_SCALE = math.sqrt(_D)

_INFO = plsc.get_sparse_core_info()
_NW = _INFO.num_cores * _INFO.num_subcores

_IDXW = 128
_KSUB = 8
_CHUNK = _IDXW * _KSUB


@functools.partial(jax.jit, static_argnames=("n_rows",))
def _gather_scale(idx2d, table, n_rows):
    b = n_rows
    b_per_w = b // _NW
    n_chunks = b_per_w // _CHUNK
    mesh = plsc.VectorSubcoreMesh(core_axis_name="c", subcore_axis_name="s")

    @functools.partial(
        pl.kernel,
        mesh=mesh,
        out_type=jax.ShapeDtypeStruct((b, _D), jnp.float32),
        scratch_types=[
            pltpu.VMEM((_KSUB, _IDXW), jnp.int32),
            pltpu.VMEM((_CHUNK, _D), jnp.float32),
            pltpu.SemaphoreType.DMA,
        ],
        compiler_params=pltpu.CompilerParams(use_tc_tiling_on_sc=False),
    )
    def k(idx_hbm, table_hbm, out_hbm, idx_v, rows_v, sem):
        wid = lax.axis_index("s") * _INFO.num_cores + lax.axis_index("c")
        base = wid * b_per_w

        @pl.loop(0, n_chunks)
        def _chunk(ci):
            cbase = base + ci * _CHUNK
            irow = pl.multiple_of(cbase // _IDXW, 8)
            pltpu.sync_copy(idx_hbm.at[pl.ds(irow, _KSUB)], idx_v)
            copies = [
                pltpu.async_copy(
                    table_hbm.at[idx_v.at[j]],
                    rows_v.at[pl.ds(j * _IDXW, _IDXW)],
                    sem,
                )
                for j in range(_KSUB)
            ]
            for c in copies:
                c.wait()

            @plsc.parallel_loop(0, _CHUNK, unroll=4)
            def _scale(r):
                for t in range(_D // 16):
                    sl = pl.ds(t * 16, 16)
                    rows_v[r, sl] = rows_v[r, sl] * _SCALE

            pltpu.sync_copy(rows_v, out_hbm.at[pl.ds(cbase, _CHUNK)])

    return k(idx2d, table)


def kernel(x, table):
    b = x.shape[0] * x.shape[1]
    idx2d = x.reshape(b // _IDXW, _IDXW).astype(jnp.int32)
    out = _gather_scale(idx2d, table, b)
    return out.reshape(x.shape[0], x.shape[1], _D)


# speedup vs baseline: 1.0599x; 1.0599x over previous
import functools
import math

import jax
import jax.numpy as jnp
from jax import lax
from jax.experimental import pallas as pl
from jax.experimental.pallas import tpu as pltpu
from jax.experimental.pallas import tpu_sc as plsc

_D = 64
_SCALE = math.sqrt(_D)

_INFO = plsc.get_sparse_core_info()
_NW = _INFO.num_cores * _INFO.num_subcores

_IDXW = 128
_KSUB = 4
_CHUNK = _IDXW * _KSUB
_NBUF = 3


@functools.partial(jax.jit, static_argnames=("n_rows",))
def _gather_scale(idx2d, table, n_rows):
    b = n_rows
    b_per_w = b // _NW
    n_steps = b_per_w // _CHUNK
    idx_rows_per_w = b_per_w // _IDXW
    mesh = plsc.VectorSubcoreMesh(core_axis_name="c", subcore_axis_name="s")

    @functools.partial(
        pl.kernel,
        mesh=mesh,
        out_type=jax.ShapeDtypeStruct((b, _D), jnp.float32),
        scratch_types=[
            pltpu.VMEM((idx_rows_per_w, _IDXW), jnp.int32),
        ]
        + [pltpu.VMEM((_CHUNK, _D), jnp.float32) for _ in range(_NBUF)]
        + [pltpu.SemaphoreType.DMA for _ in range(2 * _NBUF)],
        compiler_params=pltpu.CompilerParams(use_tc_tiling_on_sc=False),
    )
    def k(idx_hbm, table_hbm, out_hbm, idx_all, *bufs_and_sems):
        rows = bufs_and_sems[:_NBUF]
        gsem = bufs_and_sems[_NBUF:2 * _NBUF]
        osem = bufs_and_sems[2 * _NBUF:]
        wid = lax.axis_index("s") * _INFO.num_cores + lax.axis_index("c")
        base = wid * b_per_w
        irow = pl.multiple_of(wid * idx_rows_per_w, 8)
        pltpu.sync_copy(idx_hbm.at[pl.ds(irow, idx_rows_per_w)], idx_all)

        gd = {}
        od = {}

        def fire_gather(s):
            i = s % _NBUF
            gd[s] = [
                pltpu.async_copy(
                    table_hbm.at[idx_all.at[_KSUB * s + j]],
                    rows[i].at[pl.ds(j * _IDXW, _IDXW)],
                    gsem[i],
                )
                for j in range(_KSUB)
            ]

        fire_gather(0)
        if n_steps > 1:
            fire_gather(1)

        for s in range(n_steps):
            i = s % _NBUF
            for d in gd.pop(s):
                d.wait()

            @plsc.parallel_loop(0, _CHUNK, unroll=4)
            def _scale(r):
                for t in range(_D // 16):
                    sl = pl.ds(t * 16, 16)
                    rows[i][r, sl] = rows[i][r, sl] * _SCALE

            od[s] = pltpu.async_copy(
                rows[i], out_hbm.at[pl.ds(base + s * _CHUNK, _CHUNK)], osem[i]
            )
            ns = s + 2
            if ns < n_steps and ns >= 2:
                prev = ns - _NBUF
                if prev >= 0:
                    od.pop(prev).wait()
                fire_gather(ns)

        for s, d in sorted(od.items()):
            d.wait()

    return k(idx2d, table)


def kernel(x, table):
    b = x.shape[0] * x.shape[1]
    idx2d = x.reshape(b // _IDXW, _IDXW).astype(jnp.int32)
    out = _gather_scale(idx2d, table, b)
    return out.reshape(x.shape[0], x.shape[1], _D)


# speedup vs baseline: 1.2301x; 1.1605x over previous
import functools
import math

import jax
import jax.numpy as jnp
from jax import lax
from jax.experimental import pallas as pl
from jax.experimental.pallas import tpu as pltpu
from jax.experimental.pallas import tpu_sc as plsc

_D = 64
_SCALE = math.sqrt(_D)

_INFO = plsc.get_sparse_core_info()
_NW = _INFO.num_cores * _INFO.num_subcores

_IDXW = 128
_KSUB = 4
_CHUNK = _IDXW * _KSUB
_NBUF = 3


@functools.partial(jax.jit, static_argnames=("n_rows",))
def _gather_scale(idx2d, table, n_rows):
    b = n_rows
    b_per_w = b // _NW
    n_steps = b_per_w // _CHUNK
    idx_rows_per_w = b_per_w // _IDXW
    mesh = plsc.VectorSubcoreMesh(core_axis_name="c", subcore_axis_name="s")

    @functools.partial(
        pl.kernel,
        mesh=mesh,
        out_type=jax.ShapeDtypeStruct((b, _D), jnp.float32),
        scratch_types=[
            pltpu.VMEM((idx_rows_per_w, _IDXW), jnp.int32),
        ]
        + [pltpu.VMEM((_CHUNK, _D), jnp.float32) for _ in range(_NBUF)]
        + [pltpu.SemaphoreType.DMA for _ in range(2 * _NBUF)],
        compiler_params=pltpu.CompilerParams(use_tc_tiling_on_sc=False),
    )
    def k(idx_hbm, table_hbm, out_hbm, idx_all, *bufs_and_sems):
        rows = bufs_and_sems[:_NBUF]
        gsem = bufs_and_sems[_NBUF:2 * _NBUF]
        osem = bufs_and_sems[2 * _NBUF:]
        wid = lax.axis_index("s") * _INFO.num_cores + lax.axis_index("c")
        base = wid * b_per_w
        irow = pl.multiple_of(wid * idx_rows_per_w, 8)
        pltpu.sync_copy(idx_hbm.at[pl.ds(irow, idx_rows_per_w)], idx_all)

        gd = {}
        od = {}

        def fire_gather(s):
            i = s % _NBUF
            gd[s] = [
                pltpu.async_copy(
                    table_hbm.at[idx_all.at[_KSUB * s + j]],
                    rows[i].at[pl.ds(j * _IDXW, _IDXW)],
                    gsem[i],
                )
                for j in range(_KSUB)
            ]

        fire_gather(0)
        if n_steps > 1:
            fire_gather(1)

        for s in range(n_steps):
            i = s % _NBUF
            for d in gd.pop(s):
                d.wait()

            @plsc.parallel_loop(0, _CHUNK, unroll=4)
            def _scale(r):
                for t in range(_D // 16):
                    sl = pl.ds(t * 16, 16)
                    rows[i][r, sl] = rows[i][r, sl] * _SCALE

            od[s] = pltpu.async_copy(
                rows[i], out_hbm.at[pl.ds(base + s * _CHUNK, _CHUNK)], osem[i]
            )
            ns = s + 2
            if ns < n_steps and ns >= 2:
                prev = ns - _NBUF
                if prev >= 0:
                    od.pop(prev).wait()
                fire_gather(ns)

        for s, d in sorted(od.items()):
            d.wait()

    return k(idx2d, table)


def kernel(x, table):
    from jax.experimental.layout import Layout, with_layout_constraint

    b = x.shape[0] * x.shape[1]
    idx2d = x.reshape(b // _IDXW, _IDXW).astype(jnp.int32)
    out = _gather_scale(idx2d, table, b)
    out = out.reshape(x.shape[0], x.shape[1], _D)
    return with_layout_constraint(
        out, Layout(major_to_minor=(0, 1, 2), tiling=((8,), (1024,)))
    )
